# Initial kernel scaffold; baseline (speedup 1.0000x reference)
import jax
import jax.numpy as jnp
from jax.experimental import pallas as pl


def kernel(rep, dec_mask_token):
    raise NotImplementedError("write your pallas kernel here")


# speedup vs baseline: 4.3872x; 4.3872x over previous
import functools

import jax
import jax.numpy as jnp
import numpy as np
from jax import lax
from jax.experimental import pallas as pl
from jax.experimental.pallas import tpu as pltpu
from jax.experimental.pallas import tpu_sc as plsc

_N = 100000
_D = 128
_NUM_REMASK = 50000


def _rotl(x, d):
    return ((x << np.uint32(d)) | (x >> np.uint32(32 - d))).astype(np.uint32)


def _threefry2x32(k1, k2, x0, x1):
    rots = [[13, 15, 26, 6], [17, 29, 16, 24]]
    ks0, ks1 = np.uint32(k1), np.uint32(k2)
    ks2 = np.uint32(ks0 ^ ks1 ^ np.uint32(0x1BD11BDA))
    x0 = (x0 + ks0).astype(np.uint32)
    x1 = (x1 + ks1).astype(np.uint32)
    pairs = [(ks1, ks2), (ks2, ks0), (ks0, ks1), (ks1, ks2), (ks2, ks0)]
    for g in range(5):
        for d in rots[g % 2]:
            x0 = (x0 + x1).astype(np.uint32)
            x1 = _rotl(x1, d)
            x1 = (x1 ^ x0).astype(np.uint32)
        a, b = pairs[g]
        x0 = (x0 + a).astype(np.uint32)
        x1 = (x1 + b + np.uint32(g + 1)).astype(np.uint32)
    return x0, x1


def _np_permutation(seed, n):
    x = np.arange(n, dtype=np.int32)
    num_rounds = int(np.ceil(3 * np.log(max(1, n)) / np.log(2**32 - 1)))
    key = (np.uint32((seed >> 32) & 0xFFFFFFFF), np.uint32(seed & 0xFFFFFFFF))
    for _ in range(num_rounds):
        b1, b2 = _threefry2x32(*key, np.zeros(2, np.uint32), np.arange(2, dtype=np.uint32))
        key, subkey = (b1[0], b2[0]), (b1[1], b2[1])
        r1, r2 = _threefry2x32(*subkey, np.zeros(n, np.uint32), np.arange(n, dtype=np.uint32))
        x = x[np.argsort(r1 ^ r2, kind="stable")]
    return x


_PERM = _np_permutation(42, _N)
_REMASK_NP = _PERM[:_NUM_REMASK]
_REKEEP_NP = _PERM[_NUM_REMASK:]

_NC, _NS = 2, 16
_NW = _NC * _NS
_C = 112
_CHUNKS = 14
_PW = _C * _CHUNKS
_PAD = _NW * _PW


def _pad_reshape(idx: np.ndarray) -> np.ndarray:
    p = np.pad(idx, (0, _PAD - idx.shape[0]), mode="edge")
    return np.ascontiguousarray(p.reshape(_NW, _CHUNKS, _C).astype(np.int32))


_RM_IDX = _pad_reshape(_REMASK_NP)
_RK_IDX = _pad_reshape(_REKEEP_NP)

_MESH = plsc.VectorSubcoreMesh(
    core_axis_name="c", subcore_axis_name="s", num_cores=_NC, num_subcores=_NS
)


@functools.partial(
    pl.kernel,
    mesh=_MESH,
    out_type=jax.ShapeDtypeStruct((_N, _D), jnp.float32),
    scratch_types=[
        pltpu.VMEM((_CHUNKS, _C), jnp.int32),
        pltpu.VMEM((_CHUNKS, _C), jnp.int32),
        pltpu.VMEM((_C,), jnp.int32),
        pltpu.VMEM((_C, _D), jnp.float32),
        pltpu.VMEM((2, _C, _D), jnp.float32),
        pltpu.SemaphoreType.DMA,
        pltpu.SemaphoreType.DMA,
        pltpu.SemaphoreType.DMA,
    ],
)
def _remask_sc(rep_hbm, tok_hbm, rk_hbm, rm_hbm, out_hbm,
               rk_idx, rm_idx, zidx, tok_rows, rows, sem_g, sem_s, sem_t):
    wid = lax.axis_index("s") * _NC + lax.axis_index("c")

    pltpu.sync_copy(rk_hbm.at[wid], rk_idx)
    pltpu.sync_copy(rm_hbm.at[wid], rm_idx)

    zero = jnp.zeros((16,), jnp.int32)
    for j in range(_C // 16):
        zidx[pl.ds(j * 16, 16)] = zero
    pltpu.async_copy(tok_hbm.at[zidx], tok_rows, sem_t).wait()

    rm_copies = [
        pltpu.async_copy(tok_rows, out_hbm.at[rm_idx.at[j]], sem_t)
        for j in range(_CHUNKS)
    ]

    gathers = [None] * _CHUNKS
    scatters = [None] * _CHUNKS
    gathers[0] = pltpu.async_copy(rep_hbm.at[rk_idx.at[0]], rows.at[0], sem_g)
    for j in range(_CHUNKS):
        b = j % 2
        if j + 1 < _CHUNKS:
            if j >= 1:
                scatters[j - 1].wait()
            gathers[j + 1] = pltpu.async_copy(
                rep_hbm.at[rk_idx.at[j + 1]], rows.at[1 - b], sem_g
            )
        gathers[j].wait()
        scatters[j] = pltpu.async_copy(
            rows.at[b], out_hbm.at[rk_idx.at[j]], sem_s
        )

    if _CHUNKS >= 2:
        scatters[_CHUNKS - 2].wait()
    scatters[_CHUNKS - 1].wait()
    for c in rm_copies:
        c.wait()


def kernel(rep, dec_mask_token):
    out = _remask_sc(rep, dec_mask_token, jnp.asarray(_RK_IDX), jnp.asarray(_RM_IDX))
    return (out, jnp.asarray(_REMASK_NP), jnp.asarray(_REKEEP_NP))
